# baseline (speedup 1.0000x reference)
import jax
import jax.numpy as jnp
from jax import lax
from jax.experimental import pallas as pl
from jax.experimental.pallas import tpu as pltpu
from jax.experimental.pallas import tpu_sc as plsc

_VOCAB = 38462
_NF = 26
_B = 16384
_NC, _NS, _L = 2, 16, 16
_BPH = _B // _NC
_BPT = _BPH // _NS
_NBUF = 2


def _sc_body(data_hbm, tab_hbm, out_hbm, part_hbm,
             subt0, subt1, slab_v, col_v, idx0, part_v, red_v, res_v,
             sh_idx,
             sem_t0, sem_t1, sem_s, sem_c0, sem_c1, sem_r):
    s = lax.axis_index("s")
    h = lax.axis_index("c")

    f0 = s
    f1 = s + _NS
    has2 = f1 < _NF

    cp_t0 = pltpu.async_copy(tab_hbm.at[f0], subt0, sem_t0)
    row0 = h * _BPH + s * _BPT
    cp_s = pltpu.async_copy(
        data_hbm.at[pl.ds(row0 * _NF, _BPT * _NF)], slab_v, sem_s)

    @pl.when(has2)
    def _():
        pltpu.async_copy(tab_hbm.at[f1], subt1, sem_t1).wait()

    cp_s.wait()

    lane26 = lax.iota(jnp.int32, _L) * _NF
    col_sems = [sem_c0, sem_c1]
    pending = [None] * _NBUF
    for f in range(_NF):
        j = f % _NBUF
        if pending[j] is not None:
            pending[j].wait()

        def depose(c, carry, f=f, j=j):
            ids = c * (_L * _NF) + lane26 + f
            col_v[j, pl.ds(c * _L, _L)] = plsc.load_gather(slab_v, [ids])
            return carry

        lax.fori_loop(0, _BPT // _L, depose, 0)
        pending[j] = pltpu.async_copy(
            col_v.at[j], sh_idx.at[pl.ds(f * _BPH + s * _BPT, _BPT)],
            col_sems[j])
    for cp in pending:
        cp.wait()
    plsc.subcore_barrier()

    pltpu.sync_copy(sh_idx.at[pl.ds(f0 * _BPH, _BPH)], idx0)
    cp_t0.wait()

    def acc0(c, carry):
        ids = idx0[pl.ds(c * _L, _L)]
        part_v[pl.ds(c * _L, _L)] = plsc.load_gather(subt0, [ids])
        return carry

    lax.fori_loop(0, _BPH // _L, acc0, 0)

    @pl.when(has2)
    def _():
        pltpu.sync_copy(sh_idx.at[pl.ds(f1 * _BPH, _BPH)], idx0)

        def acc1(c, carry):
            ids = idx0[pl.ds(c * _L, _L)]
            part_v[pl.ds(c * _L, _L)] = (
                part_v[pl.ds(c * _L, _L)] + plsc.load_gather(subt1, [ids]))
            return carry

        lax.fori_loop(0, _BPH // _L, acc1, 0)

    pltpu.sync_copy(part_v, part_hbm.at[pl.ds((h * _NS + s) * _BPH, _BPH)])
    plsc.subcore_barrier()
    reads = []
    for t in range(_NS):
        reads.append(pltpu.async_copy(
            part_hbm.at[pl.ds((h * _NS + t) * _BPH + s * _BPT, _BPT)],
            red_v.at[t], sem_r))
    for cp in reads:
        cp.wait()

    def red(c, carry):
        acc = red_v[0, pl.ds(c * _L, _L)]
        for t in range(1, _NS):
            acc = acc + red_v[t, pl.ds(c * _L, _L)]
        res_v[pl.ds(c * _L, _L)] = acc
        return carry

    lax.fori_loop(0, _BPT // _L, red, 0)
    pltpu.sync_copy(res_v, out_hbm.at[pl.ds(h * _BPH + s * _BPT, _BPT)])


_sc_call = pl.kernel(
    _sc_body,
    out_type=(
        jax.ShapeDtypeStruct((_B,), jnp.float32),
        jax.ShapeDtypeStruct((_NC * _NS * _BPH,), jnp.float32),
    ),
    mesh=plsc.VectorSubcoreMesh(
        core_axis_name="c", subcore_axis_name="s",
        num_cores=_NC, num_subcores=_NS,
    ),
    scratch_types=[
        pltpu.VMEM((_VOCAB,), jnp.float32),
        pltpu.VMEM((_VOCAB,), jnp.float32),
        pltpu.VMEM((_BPT * _NF,), jnp.int32),
        pltpu.VMEM((_NBUF, _BPT), jnp.int32),
        pltpu.VMEM((_BPH,), jnp.int32),
        pltpu.VMEM((_BPH,), jnp.float32),
        pltpu.VMEM((_NS, _BPT), jnp.float32),
        pltpu.VMEM((_BPT,), jnp.float32),
        pltpu.VMEM_SHARED((_NF * _BPH,), jnp.int32),
        pltpu.SemaphoreType.DMA,
        pltpu.SemaphoreType.DMA,
        pltpu.SemaphoreType.DMA,
        pltpu.SemaphoreType.DMA,
        pltpu.SemaphoreType.DMA,
        pltpu.SemaphoreType.DMA,
    ],
    compiler_params=pltpu.CompilerParams(
        needs_layout_passes=False, use_tc_tiling_on_sc=False),
)


def kernel(data, table, bias):
    dataf = data.astype(jnp.int32).reshape(_B * _NF)
    tab2 = table.reshape(_NF, _VOCAB)
    out, _ = _sc_call(dataf, tab2)
    return out.reshape(_B, 1) + bias
